# Initial kernel scaffold; baseline (speedup 1.0000x reference)
import jax
import jax.numpy as jnp
from jax.experimental import pallas as pl


def kernel(x, edge_index, fc0_W, fc0_b, gW0, gb0, gW1, gb1, gW2, gb2, gW3, gb3, out_W, out_b):
    raise NotImplementedError("write your pallas kernel here")


# speedup vs baseline: 9.6259x; 9.6259x over previous
import functools

import jax
import jax.numpy as jnp
from jax import lax
from jax.experimental import pallas as pl
from jax.experimental.pallas import tpu as pltpu
from jax.experimental.pallas import tpu_sc as plsc

_N = 10000
_D = 128
_C = 40
_NP = 10240
_BLK = 1024
_GRID = _NP // _BLK
_E = 320000
_NW = 32
_CH = 128
_NCHK = 79
_EP = _NW * _NCHK * _CH

_mesh = plsc.VectorSubcoreMesh(core_axis_name="c", subcore_axis_name="s")


def _sc_degree_body(dst_hbm, zeros16_hbm, ones_hbm, out_hbm, dst_v, ones_v, acc_sh):
    c = lax.axis_index("c")
    s = lax.axis_index("s")
    tid = c * 16 + s

    @pl.when(s == 0)
    def _():
        pltpu.sync_copy(zeros16_hbm, acc_sh)

    pltpu.sync_copy(dst_hbm.at[tid], dst_v)
    pltpu.sync_copy(ones_hbm, ones_v)
    plsc.subcore_barrier()

    def body(j, carry):
        pltpu.sync_copy(ones_v, acc_sh.at[dst_v.at[j]], add=True)
        return carry

    lax.fori_loop(0, _NCHK, body, 0)
    plsc.subcore_barrier()

    @pl.when(s == 0)
    def _():
        pltpu.sync_copy(acc_sh, out_hbm.at[c])


_sc_degree = pl.kernel(
    _sc_degree_body,
    out_type=jax.ShapeDtypeStruct((2, _NP, _D), jnp.float32),
    mesh=_mesh,
    scratch_types=[
        pltpu.VMEM((_NCHK, _CH), jnp.int32),
        pltpu.VMEM((_CH, _D), jnp.float32),
        pltpu.VMEM_SHARED((_NP, _D), jnp.float32),
    ],
)


def _sc_aggregate_body(hs_hbm, src_hbm, dst_hbm, zeros_hbm, out_hbm,
                       src_v, dst_v, rows_v, acc_sh, gsem):
    c = lax.axis_index("c")
    s = lax.axis_index("s")
    tid = c * 16 + s

    @pl.when(s == 0)
    def _():
        pltpu.sync_copy(zeros_hbm, acc_sh)

    pltpu.sync_copy(src_hbm.at[tid], src_v)
    pltpu.sync_copy(dst_hbm.at[tid], dst_v)
    plsc.subcore_barrier()

    def body(j, carry):
        pltpu.async_copy(hs_hbm.at[src_v.at[j]], rows_v, gsem).wait()
        pltpu.sync_copy(rows_v, acc_sh.at[dst_v.at[j]], add=True)
        return carry

    lax.fori_loop(0, _NCHK, body, 0)
    plsc.subcore_barrier()

    @pl.when(s == 0)
    def _():
        pltpu.sync_copy(acc_sh, out_hbm.at[c])


_sc_aggregate = pl.kernel(
    _sc_aggregate_body,
    out_type=jax.ShapeDtypeStruct((2, _NP, _D), jnp.float32),
    mesh=_mesh,
    scratch_types=[
        pltpu.VMEM((_NCHK, _CH), jnp.int32),
        pltpu.VMEM((_NCHK, _CH), jnp.int32),
        pltpu.VMEM((_CH, _D), jnp.float32),
        pltpu.VMEM_SHARED((_NP, _D), jnp.float32),
        pltpu.SemaphoreType.DMA,
    ],
)


def _row_spec(last):
    return pl.BlockSpec((_BLK, last), lambda i: (i, 0))


def _full_spec(shape):
    nd = len(shape)
    return pl.BlockSpec(shape, lambda i: (0,) * nd)


def _tc_pre_body(x_ref, fc0w_ref, fc0b_ref, gw0_ref, degp_ref,
                 s1_ref, hs_ref, dinv_ref):
    i = pl.program_id(0)
    xb = x_ref[...]
    s1 = jnp.dot(xb, fc0w_ref[...], preferred_element_type=jnp.float32)
    s1 = s1 + fc0b_ref[...]
    deg = degp_ref[0][:, :16] + degp_ref[1][:, :16] + 1.0
    rows = lax.broadcasted_iota(jnp.int32, (_BLK, 16), 0) + i * _BLK
    dinv = lax.rsqrt(deg) * (rows < _N).astype(jnp.float32)
    h = jnp.dot(s1, gw0_ref[...], preferred_element_type=jnp.float32)
    s1_ref[...] = s1
    dinv_ref[...] = dinv
    hs_ref[...] = h * dinv[:, 0:1]


_tc_pre_in_specs = [
    _row_spec(_D),
    _full_spec((_D, _D)),
    _full_spec((1, _D)),
    _full_spec((_D, _D)),
    pl.BlockSpec((2, _BLK, _D), lambda i: (0, i, 0)),
]
_tc_pre_out_specs = [_row_spec(_D), _row_spec(_D), _row_spec(16)]
_tc_pre_out_shape = [
    jax.ShapeDtypeStruct((_NP, _D), jnp.float32),
    jax.ShapeDtypeStruct((_NP, _D), jnp.float32),
    jax.ShapeDtypeStruct((_NP, 16), jnp.float32),
]
_tc_pre = pl.pallas_call(
    _tc_pre_body,
    grid=(_GRID,),
    in_specs=_tc_pre_in_specs,
    out_specs=_tc_pre_out_specs,
    out_shape=_tc_pre_out_shape,
)


def _tc_layer_parts(has_prev: bool, final: bool):
    def body(*refs):
        refs = list(refs)
        ep_ref = refs.pop(0)
        hs_ref = refs.pop(0)
        dinv_ref = refs.pop(0)
        s1_ref = refs.pop(0)
        prev_ref = refs.pop(0) if has_prev else None
        b_ref = refs.pop(0)
        w_ref = refs.pop(0)
        b2_ref = refs.pop(0) if final else None
        dinv = dinv_ref[:, 0:1]
        es = ep_ref[0] + ep_ref[1] + hs_ref[...]
        e = jax.nn.relu(es * dinv + b_ref[...] + s1_ref[...])
        comb = prev_ref[...] + e if has_prev else e
        h = jnp.dot(comb, w_ref[...], preferred_element_type=jnp.float32)
        if final:
            refs[0][...] = h + b2_ref[...]
        else:
            refs[0][...] = e
            refs[1][...] = h * dinv

    in_specs = [
        pl.BlockSpec((2, _BLK, _D), lambda i: (0, i, 0)),
        _row_spec(_D),
        _row_spec(16),
        _row_spec(_D),
    ]
    if has_prev:
        in_specs.append(_row_spec(_D))
    in_specs += [_full_spec((1, _D)), _full_spec((_D, _D))]
    if final:
        in_specs.append(_full_spec((1, _D)))
        out_specs = [_row_spec(_D)]
        out_shape = [jax.ShapeDtypeStruct((_NP, _D), jnp.float32)]
    else:
        out_specs = [_row_spec(_D), _row_spec(_D)]
        out_shape = [jax.ShapeDtypeStruct((_NP, _D), jnp.float32),
                     jax.ShapeDtypeStruct((_NP, _D), jnp.float32)]
    return body, in_specs, out_specs, out_shape


def _make_tc_layer(has_prev: bool, final: bool):
    body, in_specs, out_specs, out_shape = _tc_layer_parts(has_prev, final)
    return pl.pallas_call(
        body, grid=(_GRID,), in_specs=in_specs,
        out_specs=out_specs, out_shape=out_shape,
    )


_tc_layer = _make_tc_layer(has_prev=False, final=False)
_tc_layer_prev = _make_tc_layer(has_prev=True, final=False)
_tc_final = _make_tc_layer(has_prev=True, final=True)


def kernel(x, edge_index, fc0_W, fc0_b, gW0, gb0, gW1, gb1,
           gW2, gb2, gW3, gb3, out_W, out_b):
    f32 = jnp.float32
    xp = jnp.zeros((_NP, _D), f32).at[:_N].set(x)
    pad = jnp.full((_EP - _E,), _N, jnp.int32)
    srcp = jnp.concatenate([edge_index[0], pad]).reshape(_NW, _NCHK, _CH)
    dstp = jnp.concatenate([edge_index[1], pad]).reshape(_NW, _NCHK, _CH)

    zeros128 = jnp.zeros((_NP, _D), f32)
    ones128 = jnp.ones((_CH, _D), f32)
    fc0_b2 = fc0_b.reshape(1, _D)
    gb0_2 = gb0.reshape(1, _D)
    gb1_2 = gb1.reshape(1, _D)
    gb2_2 = gb2.reshape(1, _D)
    gb3_2 = gb3.reshape(1, _D)
    out_Wp = jnp.zeros((_D, _D), f32).at[:, :_C].set(out_W)
    out_b2 = jnp.zeros((1, _D), f32).at[0, :_C].set(out_b)

    degp = _sc_degree(dstp, zeros128, ones128)
    s1, hs1, dinv = _tc_pre(xp, fc0_W, fc0_b2, gW0, degp)

    ep1 = _sc_aggregate(hs1, srcp, dstp, zeros128)
    e0, hs2 = _tc_layer(ep1, hs1, dinv, s1, gb0_2, gW1)

    ep2 = _sc_aggregate(hs2, srcp, dstp, zeros128)
    _, hs3 = _tc_layer_prev(ep2, hs2, dinv, s1, e0, gb1_2, gW2)

    ep3 = _sc_aggregate(hs3, srcp, dstp, zeros128)
    e0b, hs4 = _tc_layer(ep3, hs3, dinv, s1, gb2_2, gW3)

    ep4 = _sc_aggregate(hs4, srcp, dstp, zeros128)
    (logits_full,) = _tc_final(ep4, hs4, dinv, s1, e0b, gb3_2, out_Wp, out_b2)

    return logits_full[:_N, :_C]


# speedup vs baseline: 10.9338x; 1.1359x over previous
import functools

import jax
import jax.numpy as jnp
from jax import lax
from jax.experimental import pallas as pl
from jax.experimental.pallas import tpu as pltpu
from jax.experimental.pallas import tpu_sc as plsc

_N = 10000
_D = 128
_C = 40
_NP = 10240
_BLK = 1024
_GRID = _NP // _BLK
_E = 320000
_NW = 32
_CH = 128
_NCHK = 79
_EP = _NW * _NCHK * _CH

_mesh = plsc.VectorSubcoreMesh(core_axis_name="c", subcore_axis_name="s")


def _sc_degree_body(dst_hbm, zeros16_hbm, ones_hbm, out_hbm, dst_v, ones_v, acc_sh):
    c = lax.axis_index("c")
    s = lax.axis_index("s")
    tid = c * 16 + s

    @pl.when(s == 0)
    def _():
        pltpu.sync_copy(zeros16_hbm, acc_sh)

    pltpu.sync_copy(dst_hbm.at[tid], dst_v)
    pltpu.sync_copy(ones_hbm, ones_v)
    plsc.subcore_barrier()

    def body(j, carry):
        pltpu.sync_copy(ones_v, acc_sh.at[dst_v.at[j]], add=True)
        return carry

    lax.fori_loop(0, _NCHK, body, 0)
    plsc.subcore_barrier()

    @pl.when(s == 0)
    def _():
        pltpu.sync_copy(acc_sh, out_hbm.at[c])


_sc_degree = pl.kernel(
    _sc_degree_body,
    out_type=jax.ShapeDtypeStruct((2, _NP, _D), jnp.float32),
    mesh=_mesh,
    scratch_types=[
        pltpu.VMEM((_NCHK, _CH), jnp.int32),
        pltpu.VMEM((_CH, _D), jnp.float32),
        pltpu.VMEM_SHARED((_NP, _D), jnp.float32),
    ],
)


def _sc_aggregate_body(hs_hbm, src_hbm, dst_hbm, zeros_hbm, out_hbm,
                       si0, si1, di0, di1, rows0, rows1, acc_sh,
                       isem0, isem1, gsem0, gsem1):
    c = lax.axis_index("c")
    s = lax.axis_index("s")
    tid = c * 16 + s

    @pl.when(s == 0)
    def _():
        pltpu.sync_copy(zeros_hbm, acc_sh)

    def prefetch(j, si, di, isem):
        pltpu.async_copy(src_hbm.at[tid, j], si.at[0], isem)
        pltpu.async_copy(dst_hbm.at[tid, j], di.at[0], isem)

    def wait_idx(si, di, isem):
        pltpu.make_async_copy(src_hbm.at[0, 0], si.at[0], isem).wait()
        pltpu.make_async_copy(dst_hbm.at[0, 0], di.at[0], isem).wait()

    def gather(si, rows, gsem):
        pltpu.async_copy(hs_hbm.at[si.at[0]], rows, gsem)

    def wait_g(si, rows, gsem):
        pltpu.make_async_copy(hs_hbm.at[si.at[0]], rows, gsem).wait()

    def scatter(di, rows):
        pltpu.sync_copy(rows, acc_sh.at[di.at[0]], add=True)

    prefetch(0, si0, di0, isem0)
    prefetch(1, si1, di1, isem1)
    wait_idx(si0, di0, isem0)
    gather(si0, rows0, gsem0)
    plsc.subcore_barrier()

    def body(i, carry):
        j = 2 * i
        wait_g(si0, rows0, gsem0)
        wait_idx(si1, di1, isem1)
        gather(si1, rows1, gsem1)
        scatter(di0, rows0)
        prefetch(j + 2, si0, di0, isem0)
        wait_g(si1, rows1, gsem1)
        wait_idx(si0, di0, isem0)
        gather(si0, rows0, gsem0)
        scatter(di1, rows1)

        @pl.when(j + 3 < _NCHK)
        def _():
            prefetch(j + 3, si1, di1, isem1)

        return carry

    lax.fori_loop(0, (_NCHK - 1) // 2, body, 0)
    wait_g(si0, rows0, gsem0)
    scatter(di0, rows0)
    plsc.subcore_barrier()

    @pl.when(s == 0)
    def _():
        pltpu.sync_copy(acc_sh, out_hbm.at[c])


_sc_aggregate = pl.kernel(
    _sc_aggregate_body,
    out_type=jax.ShapeDtypeStruct((2, _NP, _D), jnp.float32),
    mesh=_mesh,
    scratch_types=[
        pltpu.VMEM((1, _CH), jnp.int32),
        pltpu.VMEM((1, _CH), jnp.int32),
        pltpu.VMEM((1, _CH), jnp.int32),
        pltpu.VMEM((1, _CH), jnp.int32),
        pltpu.VMEM((_CH, _D), jnp.float32),
        pltpu.VMEM((_CH, _D), jnp.float32),
        pltpu.VMEM_SHARED((_NP, _D), jnp.float32),
        pltpu.SemaphoreType.DMA,
        pltpu.SemaphoreType.DMA,
        pltpu.SemaphoreType.DMA,
        pltpu.SemaphoreType.DMA,
    ],
)


def _row_spec(last):
    return pl.BlockSpec((_BLK, last), lambda i: (i, 0))


def _full_spec(shape):
    nd = len(shape)
    return pl.BlockSpec(shape, lambda i: (0,) * nd)


def _tc_pre_body(x_ref, fc0w_ref, fc0b_ref, gw0_ref, degp_ref,
                 s1_ref, hs_ref, dinv_ref):
    i = pl.program_id(0)
    xb = x_ref[...]
    s1 = jnp.dot(xb, fc0w_ref[...], preferred_element_type=jnp.float32)
    s1 = s1 + fc0b_ref[...]
    deg = degp_ref[0][:, :16] + degp_ref[1][:, :16] + 1.0
    rows = lax.broadcasted_iota(jnp.int32, (_BLK, 16), 0) + i * _BLK
    dinv = lax.rsqrt(deg) * (rows < _N).astype(jnp.float32)
    h = jnp.dot(s1, gw0_ref[...], preferred_element_type=jnp.float32)
    s1_ref[...] = s1
    dinv_ref[...] = dinv
    hs_ref[...] = h * dinv[:, 0:1]


_tc_pre_in_specs = [
    _row_spec(_D),
    _full_spec((_D, _D)),
    _full_spec((1, _D)),
    _full_spec((_D, _D)),
    pl.BlockSpec((2, _BLK, _D), lambda i: (0, i, 0)),
]
_tc_pre_out_specs = [_row_spec(_D), _row_spec(_D), _row_spec(16)]
_tc_pre_out_shape = [
    jax.ShapeDtypeStruct((_NP, _D), jnp.float32),
    jax.ShapeDtypeStruct((_NP, _D), jnp.float32),
    jax.ShapeDtypeStruct((_NP, 16), jnp.float32),
]
_tc_pre = pl.pallas_call(
    _tc_pre_body,
    grid=(_GRID,),
    in_specs=_tc_pre_in_specs,
    out_specs=_tc_pre_out_specs,
    out_shape=_tc_pre_out_shape,
)


def _tc_layer_parts(has_prev: bool, final: bool):
    def body(*refs):
        refs = list(refs)
        ep_ref = refs.pop(0)
        hs_ref = refs.pop(0)
        dinv_ref = refs.pop(0)
        s1_ref = refs.pop(0)
        prev_ref = refs.pop(0) if has_prev else None
        b_ref = refs.pop(0)
        w_ref = refs.pop(0)
        b2_ref = refs.pop(0) if final else None
        dinv = dinv_ref[:, 0:1]
        es = ep_ref[0] + ep_ref[1] + hs_ref[...]
        e = jax.nn.relu(es * dinv + b_ref[...] + s1_ref[...])
        comb = prev_ref[...] + e if has_prev else e
        h = jnp.dot(comb, w_ref[...], preferred_element_type=jnp.float32)
        if final:
            refs[0][...] = h + b2_ref[...]
        else:
            refs[0][...] = e
            refs[1][...] = h * dinv

    in_specs = [
        pl.BlockSpec((2, _BLK, _D), lambda i: (0, i, 0)),
        _row_spec(_D),
        _row_spec(16),
        _row_spec(_D),
    ]
    if has_prev:
        in_specs.append(_row_spec(_D))
    in_specs += [_full_spec((1, _D)), _full_spec((_D, _D))]
    if final:
        in_specs.append(_full_spec((1, _D)))
        out_specs = [_row_spec(_D)]
        out_shape = [jax.ShapeDtypeStruct((_NP, _D), jnp.float32)]
    else:
        out_specs = [_row_spec(_D), _row_spec(_D)]
        out_shape = [jax.ShapeDtypeStruct((_NP, _D), jnp.float32),
                     jax.ShapeDtypeStruct((_NP, _D), jnp.float32)]
    return body, in_specs, out_specs, out_shape


def _make_tc_layer(has_prev: bool, final: bool):
    body, in_specs, out_specs, out_shape = _tc_layer_parts(has_prev, final)
    return pl.pallas_call(
        body, grid=(_GRID,), in_specs=in_specs,
        out_specs=out_specs, out_shape=out_shape,
    )


_tc_layer = _make_tc_layer(has_prev=False, final=False)
_tc_layer_prev = _make_tc_layer(has_prev=True, final=False)
_tc_final = _make_tc_layer(has_prev=True, final=True)


def kernel(x, edge_index, fc0_W, fc0_b, gW0, gb0, gW1, gb1,
           gW2, gb2, gW3, gb3, out_W, out_b):
    f32 = jnp.float32
    xp = jnp.zeros((_NP, _D), f32).at[:_N].set(x)
    pad = jnp.full((_EP - _E,), _N, jnp.int32)
    srcp = jnp.concatenate([edge_index[0], pad]).reshape(_NW, _NCHK, _CH)
    dstp = jnp.concatenate([edge_index[1], pad]).reshape(_NW, _NCHK, _CH)

    zeros128 = jnp.zeros((_NP, _D), f32)
    ones128 = jnp.ones((_CH, _D), f32)
    fc0_b2 = fc0_b.reshape(1, _D)
    gb0_2 = gb0.reshape(1, _D)
    gb1_2 = gb1.reshape(1, _D)
    gb2_2 = gb2.reshape(1, _D)
    gb3_2 = gb3.reshape(1, _D)
    out_Wp = jnp.zeros((_D, _D), f32).at[:, :_C].set(out_W)
    out_b2 = jnp.zeros((1, _D), f32).at[0, :_C].set(out_b)

    degp = _sc_degree(dstp, zeros128, ones128)
    s1, hs1, dinv = _tc_pre(xp, fc0_W, fc0_b2, gW0, degp)

    ep1 = _sc_aggregate(hs1, srcp, dstp, zeros128)
    e0, hs2 = _tc_layer(ep1, hs1, dinv, s1, gb0_2, gW1)

    ep2 = _sc_aggregate(hs2, srcp, dstp, zeros128)
    _, hs3 = _tc_layer_prev(ep2, hs2, dinv, s1, e0, gb1_2, gW2)

    ep3 = _sc_aggregate(hs3, srcp, dstp, zeros128)
    e0b, hs4 = _tc_layer(ep3, hs3, dinv, s1, gb2_2, gW3)

    ep4 = _sc_aggregate(hs4, srcp, dstp, zeros128)
    (logits_full,) = _tc_final(ep4, hs4, dinv, s1, e0b, gb3_2, out_Wp, out_b2)

    return logits_full[:_N, :_C]


# speedup vs baseline: 11.8533x; 1.0841x over previous
import functools

import jax
import jax.numpy as jnp
from jax import lax
from jax.experimental import pallas as pl
from jax.experimental.pallas import tpu as pltpu
from jax.experimental.pallas import tpu_sc as plsc

_N = 10000
_D = 128
_C = 40
_NP = 10240
_BLK = 1024
_GRID = _NP // _BLK
_E = 320000
_NW = 32
_CH = 128
_NCHK = 79
_K0 = 105
_K1 = 53
_EP = _NW * _NCHK * _CH

_mesh = plsc.VectorSubcoreMesh(core_axis_name="c", subcore_axis_name="s")


def _sc_degree_body(dst_hbm, zeros16_hbm, ones_hbm, out_hbm, dst_v, ones_v, acc_sh):
    c = lax.axis_index("c")
    s = lax.axis_index("s")
    tid = c * 16 + s

    @pl.when(s == 0)
    def _():
        pltpu.sync_copy(zeros16_hbm, acc_sh)

    pltpu.sync_copy(dst_hbm.at[tid], dst_v)
    pltpu.sync_copy(ones_hbm, ones_v)
    plsc.subcore_barrier()

    def body(j, carry):
        pltpu.sync_copy(ones_v, acc_sh.at[dst_v.at[j]], add=True)
        return carry

    lax.fori_loop(0, _NCHK, body, 0)
    plsc.subcore_barrier()

    @pl.when(s == 0)
    def _():
        pltpu.sync_copy(acc_sh, out_hbm.at[c])


_sc_degree = pl.kernel(
    _sc_degree_body,
    out_type=jax.ShapeDtypeStruct((2, _NP, _D), jnp.float32),
    mesh=_mesh,
    scratch_types=[
        pltpu.VMEM((_NCHK, _CH), jnp.int32),
        pltpu.VMEM((_CH, _D), jnp.float32),
        pltpu.VMEM_SHARED((_NP, _D), jnp.float32),
    ],
)


def _sc_aggregate_body(hs_hbm, src_hbm, dst_hbm, zeros_hbm, out_hbm,
                       si0, si1, di0, di1, rows0, rows1, acc_sh,
                       isem0, isem1, gsem0, gsem1):
    c = lax.axis_index("c")
    s = lax.axis_index("s")
    cnt = jnp.where(c == 0, _K0, _K1)
    base = jnp.where(c == 0, s * _K0, 16 * _K0 + s * _K1)

    @pl.when(s == 0)
    def _():
        pltpu.sync_copy(zeros_hbm, acc_sh)

    def prefetch(j, si, di, isem):
        pltpu.async_copy(src_hbm.at[base + j], si.at[0], isem)
        pltpu.async_copy(dst_hbm.at[base + j], di.at[0], isem)

    def wait_idx(si, di, isem):
        pltpu.make_async_copy(src_hbm.at[0], si.at[0], isem).wait()
        pltpu.make_async_copy(dst_hbm.at[0], di.at[0], isem).wait()

    def gather(si, rows, gsem):
        pltpu.async_copy(hs_hbm.at[si.at[0]], rows, gsem)

    def wait_g(si, rows, gsem):
        pltpu.make_async_copy(hs_hbm.at[si.at[0]], rows, gsem).wait()

    def scatter(di, rows):
        pltpu.sync_copy(rows, acc_sh.at[di.at[0]], add=True)

    prefetch(0, si0, di0, isem0)
    prefetch(1, si1, di1, isem1)
    wait_idx(si0, di0, isem0)
    gather(si0, rows0, gsem0)
    plsc.subcore_barrier()

    def body(i, carry):
        j = 2 * i
        wait_g(si0, rows0, gsem0)
        wait_idx(si1, di1, isem1)
        gather(si1, rows1, gsem1)
        scatter(di0, rows0)
        prefetch(j + 2, si0, di0, isem0)
        wait_g(si1, rows1, gsem1)
        wait_idx(si0, di0, isem0)
        gather(si0, rows0, gsem0)
        scatter(di1, rows1)

        @pl.when(j + 3 < cnt)
        def _():
            prefetch(j + 3, si1, di1, isem1)

        return carry

    lax.fori_loop(0, (cnt - 1) // 2, body, 0)
    wait_g(si0, rows0, gsem0)
    scatter(di0, rows0)
    plsc.subcore_barrier()

    @pl.when(s == 0)
    def _():
        pltpu.sync_copy(acc_sh, out_hbm.at[c])


_sc_aggregate = pl.kernel(
    _sc_aggregate_body,
    out_type=jax.ShapeDtypeStruct((2, _NP, _D), jnp.float32),
    mesh=_mesh,
    scratch_types=[
        pltpu.VMEM((1, _CH), jnp.int32),
        pltpu.VMEM((1, _CH), jnp.int32),
        pltpu.VMEM((1, _CH), jnp.int32),
        pltpu.VMEM((1, _CH), jnp.int32),
        pltpu.VMEM((_CH, _D), jnp.float32),
        pltpu.VMEM((_CH, _D), jnp.float32),
        pltpu.VMEM_SHARED((_NP, _D), jnp.float32),
        pltpu.SemaphoreType.DMA,
        pltpu.SemaphoreType.DMA,
        pltpu.SemaphoreType.DMA,
        pltpu.SemaphoreType.DMA,
    ],
)


def _row_spec(last):
    return pl.BlockSpec((_BLK, last), lambda i: (i, 0))


def _full_spec(shape):
    nd = len(shape)
    return pl.BlockSpec(shape, lambda i: (0,) * nd)


def _tc_pre_body(x_ref, fc0w_ref, fc0b_ref, gw0_ref, degp_ref,
                 s1_ref, hs_ref, dinv_ref):
    i = pl.program_id(0)
    xb = x_ref[...]
    s1 = jnp.dot(xb, fc0w_ref[...], preferred_element_type=jnp.float32)
    s1 = s1 + fc0b_ref[...]
    deg = degp_ref[0][:, :16] + degp_ref[1][:, :16] + 1.0
    rows = lax.broadcasted_iota(jnp.int32, (_BLK, 16), 0) + i * _BLK
    dinv = lax.rsqrt(deg) * (rows < _N).astype(jnp.float32)
    h = jnp.dot(s1, gw0_ref[...], preferred_element_type=jnp.float32)
    s1_ref[...] = s1
    dinv_ref[...] = dinv
    hs_ref[...] = h * dinv[:, 0:1]


_tc_pre_in_specs = [
    _row_spec(_D),
    _full_spec((_D, _D)),
    _full_spec((1, _D)),
    _full_spec((_D, _D)),
    pl.BlockSpec((2, _BLK, _D), lambda i: (0, i, 0)),
]
_tc_pre_out_specs = [_row_spec(_D), _row_spec(_D), _row_spec(16)]
_tc_pre_out_shape = [
    jax.ShapeDtypeStruct((_NP, _D), jnp.float32),
    jax.ShapeDtypeStruct((_NP, _D), jnp.float32),
    jax.ShapeDtypeStruct((_NP, 16), jnp.float32),
]
_tc_pre = pl.pallas_call(
    _tc_pre_body,
    grid=(_GRID,),
    in_specs=_tc_pre_in_specs,
    out_specs=_tc_pre_out_specs,
    out_shape=_tc_pre_out_shape,
)


def _tc_layer_parts(has_prev: bool, final: bool):
    def body(*refs):
        refs = list(refs)
        ep_ref = refs.pop(0)
        hs_ref = refs.pop(0)
        dinv_ref = refs.pop(0)
        s1_ref = refs.pop(0)
        prev_ref = refs.pop(0) if has_prev else None
        b_ref = refs.pop(0)
        w_ref = refs.pop(0)
        b2_ref = refs.pop(0) if final else None
        dinv = dinv_ref[:, 0:1]
        es = ep_ref[0] + ep_ref[1] + hs_ref[...]
        e = jax.nn.relu(es * dinv + b_ref[...] + s1_ref[...])
        comb = prev_ref[...] + e if has_prev else e
        h = jnp.dot(comb, w_ref[...], preferred_element_type=jnp.float32)
        if final:
            refs[0][...] = h + b2_ref[...]
        else:
            refs[0][...] = e
            refs[1][...] = h * dinv

    in_specs = [
        pl.BlockSpec((2, _BLK, _D), lambda i: (0, i, 0)),
        _row_spec(_D),
        _row_spec(16),
        _row_spec(_D),
    ]
    if has_prev:
        in_specs.append(_row_spec(_D))
    in_specs += [_full_spec((1, _D)), _full_spec((_D, _D))]
    if final:
        in_specs.append(_full_spec((1, _D)))
        out_specs = [_row_spec(_D)]
        out_shape = [jax.ShapeDtypeStruct((_NP, _D), jnp.float32)]
    else:
        out_specs = [_row_spec(_D), _row_spec(_D)]
        out_shape = [jax.ShapeDtypeStruct((_NP, _D), jnp.float32),
                     jax.ShapeDtypeStruct((_NP, _D), jnp.float32)]
    return body, in_specs, out_specs, out_shape


def _make_tc_layer(has_prev: bool, final: bool):
    body, in_specs, out_specs, out_shape = _tc_layer_parts(has_prev, final)
    return pl.pallas_call(
        body, grid=(_GRID,), in_specs=in_specs,
        out_specs=out_specs, out_shape=out_shape,
    )


_tc_layer = _make_tc_layer(has_prev=False, final=False)
_tc_layer_prev = _make_tc_layer(has_prev=True, final=False)
_tc_final = _make_tc_layer(has_prev=True, final=True)


def kernel(x, edge_index, fc0_W, fc0_b, gW0, gb0, gW1, gb1,
           gW2, gb2, gW3, gb3, out_W, out_b):
    f32 = jnp.float32
    xp = jnp.zeros((_NP, _D), f32).at[:_N].set(x)
    pad = jnp.full((_EP - _E,), _N, jnp.int32)
    srcp = jnp.concatenate([edge_index[0], pad]).reshape(_EP // _CH, _CH)
    dstp = jnp.concatenate([edge_index[1], pad]).reshape(_EP // _CH, _CH)
    dstp3 = dstp.reshape(_NW, _NCHK, _CH)

    zeros128 = jnp.zeros((_NP, _D), f32)
    ones128 = jnp.ones((_CH, _D), f32)
    fc0_b2 = fc0_b.reshape(1, _D)
    gb0_2 = gb0.reshape(1, _D)
    gb1_2 = gb1.reshape(1, _D)
    gb2_2 = gb2.reshape(1, _D)
    gb3_2 = gb3.reshape(1, _D)
    out_Wp = jnp.zeros((_D, _D), f32).at[:, :_C].set(out_W)
    out_b2 = jnp.zeros((1, _D), f32).at[0, :_C].set(out_b)

    degp = _sc_degree(dstp3, zeros128, ones128)
    s1, hs1, dinv = _tc_pre(xp, fc0_W, fc0_b2, gW0, degp)

    ep1 = _sc_aggregate(hs1, srcp, dstp, zeros128)
    e0, hs2 = _tc_layer(ep1, hs1, dinv, s1, gb0_2, gW1)

    ep2 = _sc_aggregate(hs2, srcp, dstp, zeros128)
    _, hs3 = _tc_layer_prev(ep2, hs2, dinv, s1, e0, gb1_2, gW2)

    ep3 = _sc_aggregate(hs3, srcp, dstp, zeros128)
    e0b, hs4 = _tc_layer(ep3, hs3, dinv, s1, gb2_2, gW3)

    ep4 = _sc_aggregate(hs4, srcp, dstp, zeros128)
    (logits_full,) = _tc_final(ep4, hs4, dinv, s1, e0b, gb3_2, out_Wp, out_b2)

    return logits_full[:_N, :_C]


# speedup vs baseline: 12.1477x; 1.0248x over previous
import functools

import jax
import jax.numpy as jnp
from jax import lax
from jax.experimental import pallas as pl
from jax.experimental.pallas import tpu as pltpu
from jax.experimental.pallas import tpu_sc as plsc

_N = 10000
_D = 128
_C = 40
_NP = 10240
_BLK = 1024
_GRID = _NP // _BLK
_E = 320000
_NW = 32
_CH = 128
_NCHK = 79
_K0 = 105
_K1 = 53
_EP = _NW * _NCHK * _CH

_mesh = plsc.VectorSubcoreMesh(core_axis_name="c", subcore_axis_name="s")


def _sc_degree_body(dst_hbm, zeros16_hbm, ones_hbm, out_hbm, dst_v, ones_v, acc_sh):
    c = lax.axis_index("c")
    s = lax.axis_index("s")
    tid = c * 16 + s

    @pl.when(s == 0)
    def _():
        pltpu.sync_copy(zeros16_hbm, acc_sh)

    pltpu.sync_copy(dst_hbm.at[tid], dst_v)
    pltpu.sync_copy(ones_hbm, ones_v)
    plsc.subcore_barrier()

    def body(j, carry):
        pltpu.sync_copy(ones_v, acc_sh.at[dst_v.at[j]], add=True)
        return carry

    lax.fori_loop(0, _NCHK, body, 0)
    plsc.subcore_barrier()

    @pl.when(s == 0)
    def _():
        pltpu.sync_copy(acc_sh, out_hbm.at[c])


_sc_degree = pl.kernel(
    _sc_degree_body,
    out_type=jax.ShapeDtypeStruct((2, _NP, _D), jnp.float32),
    mesh=_mesh,
    scratch_types=[
        pltpu.VMEM((_NCHK, _CH), jnp.int32),
        pltpu.VMEM((_CH, _D), jnp.float32),
        pltpu.VMEM_SHARED((_NP, _D), jnp.float32),
    ],
)


def _sc_aggregate_body(hs_hbm, src_hbm, dst_hbm, zeros_hbm, out_hbm,
                       si0, si1, di0, di1, rows0, rows1, acc_sh,
                       isem0, isem1, gsem0, gsem1):
    c = lax.axis_index("c")
    s = lax.axis_index("s")
    cnt = jnp.where(c == 0, _K0, _K1)
    base = jnp.where(c == 0, s * _K0, 16 * _K0 + s * _K1)
    rslice = pl.ds(s * (_NP // 16), _NP // 16)

    @pl.when(c == 0)
    def _():
        pltpu.sync_copy(hs_hbm.at[rslice], acc_sh.at[rslice])

    @pl.when(c == 1)
    def _():
        pltpu.sync_copy(zeros_hbm.at[rslice], acc_sh.at[rslice])

    def prefetch(j, si, di, isem):
        pltpu.async_copy(src_hbm.at[base + j], si.at[0], isem)
        pltpu.async_copy(dst_hbm.at[base + j], di.at[0], isem)

    def wait_idx(si, di, isem):
        pltpu.make_async_copy(src_hbm.at[0], si.at[0], isem).wait()
        pltpu.make_async_copy(dst_hbm.at[0], di.at[0], isem).wait()

    def gather(si, rows, gsem):
        pltpu.async_copy(hs_hbm.at[si.at[0]], rows, gsem)

    def wait_g(si, rows, gsem):
        pltpu.make_async_copy(hs_hbm.at[si.at[0]], rows, gsem).wait()

    def scatter(di, rows):
        pltpu.sync_copy(rows, acc_sh.at[di.at[0]], add=True)

    prefetch(0, si0, di0, isem0)
    prefetch(1, si1, di1, isem1)
    wait_idx(si0, di0, isem0)
    gather(si0, rows0, gsem0)
    plsc.subcore_barrier()

    def body(i, carry):
        j = 2 * i
        wait_g(si0, rows0, gsem0)
        wait_idx(si1, di1, isem1)
        gather(si1, rows1, gsem1)
        scatter(di0, rows0)
        prefetch(j + 2, si0, di0, isem0)
        wait_g(si1, rows1, gsem1)
        wait_idx(si0, di0, isem0)
        gather(si0, rows0, gsem0)
        scatter(di1, rows1)

        @pl.when(j + 3 < cnt)
        def _():
            prefetch(j + 3, si1, di1, isem1)

        return carry

    lax.fori_loop(0, (cnt - 1) // 2, body, 0)
    wait_g(si0, rows0, gsem0)
    scatter(di0, rows0)
    plsc.subcore_barrier()
    pltpu.sync_copy(acc_sh.at[rslice], out_hbm.at[c].at[rslice])


_sc_aggregate = pl.kernel(
    _sc_aggregate_body,
    out_type=jax.ShapeDtypeStruct((2, _NP, _D), jnp.float32),
    mesh=_mesh,
    scratch_types=[
        pltpu.VMEM((1, _CH), jnp.int32),
        pltpu.VMEM((1, _CH), jnp.int32),
        pltpu.VMEM((1, _CH), jnp.int32),
        pltpu.VMEM((1, _CH), jnp.int32),
        pltpu.VMEM((_CH, _D), jnp.float32),
        pltpu.VMEM((_CH, _D), jnp.float32),
        pltpu.VMEM_SHARED((_NP, _D), jnp.float32),
        pltpu.SemaphoreType.DMA,
        pltpu.SemaphoreType.DMA,
        pltpu.SemaphoreType.DMA,
        pltpu.SemaphoreType.DMA,
    ],
)


def _row_spec(last):
    return pl.BlockSpec((_BLK, last), lambda i: (i, 0))


def _full_spec(shape):
    nd = len(shape)
    return pl.BlockSpec(shape, lambda i: (0,) * nd)


def _tc_pre_body(x_ref, fc0w_ref, fc0b_ref, gw0_ref, degp_ref,
                 s1_ref, hs_ref, dinv_ref):
    i = pl.program_id(0)
    xb = x_ref[...]
    s1 = jnp.dot(xb, fc0w_ref[...], preferred_element_type=jnp.float32)
    s1 = s1 + fc0b_ref[...]
    deg = degp_ref[0][:, :16] + degp_ref[1][:, :16] + 1.0
    rows = lax.broadcasted_iota(jnp.int32, (_BLK, 16), 0) + i * _BLK
    dinv = lax.rsqrt(deg) * (rows < _N).astype(jnp.float32)
    h = jnp.dot(s1, gw0_ref[...], preferred_element_type=jnp.float32)
    s1_ref[...] = s1
    dinv_ref[...] = dinv
    hs_ref[...] = h * dinv[:, 0:1]


_tc_pre_in_specs = [
    _row_spec(_D),
    _full_spec((_D, _D)),
    _full_spec((1, _D)),
    _full_spec((_D, _D)),
    pl.BlockSpec((2, _BLK, _D), lambda i: (0, i, 0)),
]
_tc_pre_out_specs = [_row_spec(_D), _row_spec(_D), _row_spec(16)]
_tc_pre_out_shape = [
    jax.ShapeDtypeStruct((_NP, _D), jnp.float32),
    jax.ShapeDtypeStruct((_NP, _D), jnp.float32),
    jax.ShapeDtypeStruct((_NP, 16), jnp.float32),
]
_tc_pre = pl.pallas_call(
    _tc_pre_body,
    grid=(_GRID,),
    in_specs=_tc_pre_in_specs,
    out_specs=_tc_pre_out_specs,
    out_shape=_tc_pre_out_shape,
)


def _tc_layer_parts(has_prev: bool, final: bool):
    def body(*refs):
        refs = list(refs)
        ep_ref = refs.pop(0)
        dinv_ref = refs.pop(0)
        s1_ref = refs.pop(0)
        prev_ref = refs.pop(0) if has_prev else None
        b_ref = refs.pop(0)
        w_ref = refs.pop(0)
        b2_ref = refs.pop(0) if final else None
        dinv = dinv_ref[:, 0:1]
        es = ep_ref[0] + ep_ref[1]
        e = jax.nn.relu(es * dinv + b_ref[...] + s1_ref[...])
        comb = prev_ref[...] + e if has_prev else e
        h = jnp.dot(comb, w_ref[...], preferred_element_type=jnp.float32)
        if final:
            refs[0][...] = h + b2_ref[...]
        else:
            refs[0][...] = e
            refs[1][...] = h * dinv

    in_specs = [
        pl.BlockSpec((2, _BLK, _D), lambda i: (0, i, 0)),
        _row_spec(16),
        _row_spec(_D),
    ]
    if has_prev:
        in_specs.append(_row_spec(_D))
    in_specs += [_full_spec((1, _D)), _full_spec((_D, _D))]
    if final:
        in_specs.append(_full_spec((1, _D)))
        out_specs = [_row_spec(_D)]
        out_shape = [jax.ShapeDtypeStruct((_NP, _D), jnp.float32)]
    else:
        out_specs = [_row_spec(_D), _row_spec(_D)]
        out_shape = [jax.ShapeDtypeStruct((_NP, _D), jnp.float32),
                     jax.ShapeDtypeStruct((_NP, _D), jnp.float32)]
    return body, in_specs, out_specs, out_shape


def _make_tc_layer(has_prev: bool, final: bool):
    body, in_specs, out_specs, out_shape = _tc_layer_parts(has_prev, final)
    return pl.pallas_call(
        body, grid=(_GRID,), in_specs=in_specs,
        out_specs=out_specs, out_shape=out_shape,
    )


_tc_layer = _make_tc_layer(has_prev=False, final=False)
_tc_layer_prev = _make_tc_layer(has_prev=True, final=False)
_tc_final = _make_tc_layer(has_prev=True, final=True)


def kernel(x, edge_index, fc0_W, fc0_b, gW0, gb0, gW1, gb1,
           gW2, gb2, gW3, gb3, out_W, out_b):
    f32 = jnp.float32
    xp = jnp.zeros((_NP, _D), f32).at[:_N].set(x)
    pad = jnp.full((_EP - _E,), _N, jnp.int32)
    srcp = jnp.concatenate([edge_index[0], pad]).reshape(_EP // _CH, _CH)
    dstp = jnp.concatenate([edge_index[1], pad]).reshape(_EP // _CH, _CH)
    dstp3 = dstp.reshape(_NW, _NCHK, _CH)

    zeros128 = jnp.zeros((_NP, _D), f32)
    ones128 = jnp.ones((_CH, _D), f32)
    fc0_b2 = fc0_b.reshape(1, _D)
    gb0_2 = gb0.reshape(1, _D)
    gb1_2 = gb1.reshape(1, _D)
    gb2_2 = gb2.reshape(1, _D)
    gb3_2 = gb3.reshape(1, _D)
    out_Wp = jnp.zeros((_D, _D), f32).at[:, :_C].set(out_W)
    out_b2 = jnp.zeros((1, _D), f32).at[0, :_C].set(out_b)

    degp = _sc_degree(dstp3, zeros128, ones128)
    s1, hs1, dinv = _tc_pre(xp, fc0_W, fc0_b2, gW0, degp)

    ep1 = _sc_aggregate(hs1, srcp, dstp, zeros128)
    e0, hs2 = _tc_layer(ep1, dinv, s1, gb0_2, gW1)

    ep2 = _sc_aggregate(hs2, srcp, dstp, zeros128)
    _, hs3 = _tc_layer_prev(ep2, dinv, s1, e0, gb1_2, gW2)

    ep3 = _sc_aggregate(hs3, srcp, dstp, zeros128)
    e0b, hs4 = _tc_layer(ep3, dinv, s1, gb2_2, gW3)

    ep4 = _sc_aggregate(hs4, srcp, dstp, zeros128)
    (logits_full,) = _tc_final(ep4, dinv, s1, e0b, gb3_2, out_Wp, out_b2)

    return logits_full[:_N, :_C]
